# Initial kernel scaffold; baseline (speedup 1.0000x reference)
import jax
import jax.numpy as jnp
from jax.experimental import pallas as pl


def kernel(xyz):
    raise NotImplementedError("write your pallas kernel here")


# speedup vs baseline: 6.1717x; 6.1717x over previous
import jax
import jax.numpy as jnp
from jax.experimental import pallas as pl
from jax.experimental.pallas import tpu as pltpu

B, N, G, M = 8, 8192, 256, 32


def _fps_body(x_ref, y_ref, z_ref, cx_ref, cy_ref, cz_ref):
    x = x_ref[...]
    y = y_ref[...]
    z = z_ref[...]
    lane = jax.lax.broadcasted_iota(jnp.int32, (B, N), 1)
    col = jax.lax.broadcasted_iota(jnp.int32, (B, G), 1)

    lastx = x[:, 0:1]
    lasty = y[:, 0:1]
    lastz = z[:, 0:1]
    zerosg = jnp.zeros((B, G), jnp.float32)
    cx = jnp.where(col == 0, lastx, zerosg)
    cy = jnp.where(col == 0, lasty, zerosg)
    cz = jnp.where(col == 0, lastz, zerosg)
    dists = jnp.full((B, N), 1e10, jnp.float32)

    def step(i, carry):
        dists, lastx, lasty, lastz, cx, cy, cz = carry
        dx = x - lastx
        dy = y - lasty
        dz = z - lastz
        d = (dx * dx + dy * dy) + dz * dz
        dists = jnp.minimum(dists, d)
        m = jnp.max(dists, axis=1, keepdims=True)
        t = jnp.where(dists == m, lane, N)
        idx = jnp.min(t, axis=1, keepdims=True)
        oh = lane == idx
        zn = jnp.zeros((B, N), jnp.float32)
        lastx = jnp.sum(jnp.where(oh, x, zn), axis=1, keepdims=True)
        lasty = jnp.sum(jnp.where(oh, y, zn), axis=1, keepdims=True)
        lastz = jnp.sum(jnp.where(oh, z, zn), axis=1, keepdims=True)
        upd = col == i
        cx = jnp.where(upd, lastx, cx)
        cy = jnp.where(upd, lasty, cy)
        cz = jnp.where(upd, lastz, cz)
        return (dists, lastx, lasty, lastz, cx, cy, cz)

    carry = (dists, lastx, lasty, lastz, cx, cy, cz)
    carry = jax.lax.fori_loop(1, G, step, carry)
    _, _, _, _, cx, cy, cz = carry
    cx_ref[...] = cx
    cy_ref[...] = cy
    cz_ref[...] = cz


def _knn_body(x_ref, y_ref, z_ref, cxT_ref, cyT_ref, czT_ref,
              nbx_ref, nby_ref, nbz_ref, d2_ref):
    x = x_ref[0]
    y = y_ref[0]
    z = z_ref[0]
    cx = cxT_ref[0]
    cy = cyT_ref[0]
    cz = czT_ref[0]

    dx = cx - x
    dy = cy - y
    dz = cz - z
    d2_ref[...] = (dx * dx + dy * dy) + dz * dz

    lane = jax.lax.broadcasted_iota(jnp.int32, (G, N), 1)
    col = jax.lax.broadcasted_iota(jnp.int32, (G, M), 1)
    inf = jnp.float32(jnp.inf)
    zgm = jnp.zeros((G, M), jnp.float32)
    zgn = jnp.zeros((G, N), jnp.float32)

    def step(s, carry):
        d2s, nbx, nby, nbz = carry
        w = d2_ref[...]
        m = jnp.min(w, axis=1, keepdims=True)
        t = jnp.where(w == m, lane, N)
        idx = jnp.min(t, axis=1, keepdims=True)
        oh = lane == idx
        d2_ref[...] = jnp.where(oh, inf, w)
        px = jnp.sum(jnp.where(oh, x, zgn), axis=1, keepdims=True)
        py = jnp.sum(jnp.where(oh, y, zgn), axis=1, keepdims=True)
        pz = jnp.sum(jnp.where(oh, z, zgn), axis=1, keepdims=True)
        upd = col == s
        d2s = jnp.where(upd, m, d2s)
        nbx = jnp.where(upd, px - cx, nbx)
        nby = jnp.where(upd, py - cy, nby)
        nbz = jnp.where(upd, pz - cz, nbz)
        return (d2s, nbx, nby, nbz)

    carry = (zgm, zgm, zgm, zgm)
    d2s, nbx, nby, nbz = jax.lax.fori_loop(0, M, step, carry)

    scale = jnp.max(jnp.sqrt(d2s + 1e-12), axis=1, keepdims=True)
    scale = jnp.maximum(scale, 1e-8)
    nbx_ref[0] = nbx / scale
    nby_ref[0] = nby / scale
    nbz_ref[0] = nbz / scale


def kernel(xyz):
    x = xyz[:, :, 0]
    y = xyz[:, :, 1]
    z = xyz[:, :, 2]

    cx, cy, cz = pl.pallas_call(
        _fps_body,
        out_shape=[jax.ShapeDtypeStruct((B, G), jnp.float32)] * 3,
    )(x, y, z)

    x3 = x.reshape(B, 1, N)
    y3 = y.reshape(B, 1, N)
    z3 = z.reshape(B, 1, N)
    cxT = cx.reshape(B, G, 1)
    cyT = cy.reshape(B, G, 1)
    czT = cz.reshape(B, G, 1)
    nbx, nby, nbz = pl.pallas_call(
        _knn_body,
        grid=(B,),
        in_specs=[pl.BlockSpec((1, 1, N), lambda b: (b, 0, 0))] * 3
        + [pl.BlockSpec((1, G, 1), lambda b: (b, 0, 0))] * 3,
        out_specs=[pl.BlockSpec((1, G, M), lambda b: (b, 0, 0))] * 3,
        out_shape=[jax.ShapeDtypeStruct((B, G, M), jnp.float32)] * 3,
        scratch_shapes=[pltpu.VMEM((G, N), jnp.float32)],
    )(x3, y3, z3, cxT, cyT, czT)

    neighborhood = jnp.stack([nbx, nby, nbz], axis=-1)
    center = jnp.stack([cx, cy, cz], axis=-1)
    return neighborhood, center


# speedup vs baseline: 10.3602x; 1.6787x over previous
import functools

import jax
import jax.numpy as jnp
from jax import lax
from jax.experimental import pallas as pl
from jax.experimental.pallas import tpu as pltpu
from jax.experimental.pallas import tpu_sc as plsc

B, N, G, M = 8, 8192, 256, 32


def _fps_body(x_ref, y_ref, z_ref, cx_ref, cy_ref, cz_ref):
    x = x_ref[...]
    y = y_ref[...]
    z = z_ref[...]
    lane = jax.lax.broadcasted_iota(jnp.int32, (B, N), 1)
    col = jax.lax.broadcasted_iota(jnp.int32, (B, G), 1)

    lastx = x[:, 0:1]
    lasty = y[:, 0:1]
    lastz = z[:, 0:1]
    zerosg = jnp.zeros((B, G), jnp.float32)
    cx = jnp.where(col == 0, lastx, zerosg)
    cy = jnp.where(col == 0, lasty, zerosg)
    cz = jnp.where(col == 0, lastz, zerosg)
    dists = jnp.full((B, N), 1e10, jnp.float32)

    def step(i, carry):
        dists, lastx, lasty, lastz, cx, cy, cz = carry
        dx = x - lastx
        dy = y - lasty
        dz = z - lastz
        d = (dx * dx + dy * dy) + dz * dz
        dists = jnp.minimum(dists, d)
        m = jnp.max(dists, axis=1, keepdims=True)
        t = jnp.where(dists == m, lane, N)
        idx = jnp.min(t, axis=1, keepdims=True)
        oh = t == idx
        zn = jnp.zeros((B, N), jnp.float32)
        lastx = jnp.sum(jnp.where(oh, x, zn), axis=1, keepdims=True)
        lasty = jnp.sum(jnp.where(oh, y, zn), axis=1, keepdims=True)
        lastz = jnp.sum(jnp.where(oh, z, zn), axis=1, keepdims=True)
        upd = col == i
        cx = jnp.where(upd, lastx, cx)
        cy = jnp.where(upd, lasty, cy)
        cz = jnp.where(upd, lastz, cz)
        return (dists, lastx, lasty, lastz, cx, cy, cz)

    carry = (dists, lastx, lasty, lastz, cx, cy, cz)
    carry = jax.lax.fori_loop(1, G, step, carry)
    _, _, _, _, cx, cy, cz = carry
    cx_ref[...] = cx
    cy_ref[...] = cy
    cz_ref[...] = cz


def _knn_body(x_ref, y_ref, z_ref, cxT_ref, cyT_ref, czT_ref,
              knn_ref, scale_ref, d2_ref):
    x = x_ref[0]
    y = y_ref[0]
    z = z_ref[0]
    cx = cxT_ref[0]
    cy = cyT_ref[0]
    cz = czT_ref[0]

    dx = cx - x
    dy = cy - y
    dz = cz - z
    d2_ref[...] = (dx * dx + dy * dy) + dz * dz

    lane = jax.lax.broadcasted_iota(jnp.int32, (G, N), 1)
    col = jax.lax.broadcasted_iota(jnp.int32, (G, M), 1)
    inf = jnp.float32(jnp.inf)
    zgm = jnp.zeros((G, M), jnp.float32)
    igm = jnp.zeros((G, M), jnp.int32)

    def step(s, carry):
        knn, d2s = carry
        w = d2_ref[...]
        m = jnp.min(w, axis=1, keepdims=True)
        t = jnp.where(w == m, lane, N)
        idx = jnp.min(t, axis=1, keepdims=True)
        d2_ref[...] = jnp.where(t == idx, inf, w)
        upd = col == s
        knn = jnp.where(upd, idx, knn)
        d2s = jnp.where(upd, m, d2s)
        return (knn, d2s)

    knn, d2s = jax.lax.fori_loop(0, M, step, (igm, zgm))

    scale = jnp.max(jnp.sqrt(d2s + 1e-12), axis=1, keepdims=True)
    scale = jnp.maximum(scale, 1e-8)
    knn_ref[0] = knn
    scale_ref[0] = scale


def _gather_body(xf, yf, zf, knnf, sf, cxf, cyf, czf,
                 ox_hbm, oy_hbm, oz_hbm,
                 xv, yv, zv, kv, sv, cxv, cyv, czv, oxv, oyv, ozv):
    wid = lax.axis_index("s") * 2 + lax.axis_index("c")
    b = wid // 4
    q = wid % 4
    gbase = b * G + q * 64
    obase = gbase * M

    pltpu.sync_copy(xf.at[pl.ds(b * N, N)], xv)
    pltpu.sync_copy(yf.at[pl.ds(b * N, N)], yv)
    pltpu.sync_copy(zf.at[pl.ds(b * N, N)], zv)
    pltpu.sync_copy(knnf.at[pl.ds(obase, 64 * M)], kv)
    pltpu.sync_copy(sf.at[pl.ds(gbase, 64)], sv)
    pltpu.sync_copy(cxf.at[pl.ds(gbase, 64)], cxv)
    pltpu.sync_copy(cyf.at[pl.ds(gbase, 64)], cyv)
    pltpu.sync_copy(czf.at[pl.ds(gbase, 64)], czv)

    lanes = lax.iota(jnp.int32, 16)
    for gg in range(4):
        cbase = gg * 16
        cxr = cxv[pl.ds(cbase, 16)]
        cyr = cyv[pl.ds(cbase, 16)]
        czr = czv[pl.ds(cbase, 16)]
        sr = sv[pl.ds(cbase, 16)]
        posb = (lanes + cbase) * M
        for j in range(M):
            pos = posb + j
            idx = plsc.load_gather(kv, [pos])
            vx = plsc.load_gather(xv, [idx])
            vy = plsc.load_gather(yv, [idx])
            vz = plsc.load_gather(zv, [idx])
            plsc.store_scatter(oxv, [pos], (vx - cxr) / sr)
            plsc.store_scatter(oyv, [pos], (vy - cyr) / sr)
            plsc.store_scatter(ozv, [pos], (vz - czr) / sr)

    pltpu.sync_copy(oxv, ox_hbm.at[pl.ds(obase, 64 * M)])
    pltpu.sync_copy(oyv, oy_hbm.at[pl.ds(obase, 64 * M)])
    pltpu.sync_copy(ozv, oz_hbm.at[pl.ds(obase, 64 * M)])


_gather_call = functools.partial(
    pl.kernel,
    out_type=[jax.ShapeDtypeStruct((B * G * M,), jnp.float32)] * 3,
    mesh=plsc.VectorSubcoreMesh(core_axis_name="c", subcore_axis_name="s"),
    compiler_params=pltpu.CompilerParams(needs_layout_passes=False),
    scratch_types=[
        pltpu.VMEM((N,), jnp.float32),
        pltpu.VMEM((N,), jnp.float32),
        pltpu.VMEM((N,), jnp.float32),
        pltpu.VMEM((64 * M,), jnp.int32),
        pltpu.VMEM((64,), jnp.float32),
        pltpu.VMEM((64,), jnp.float32),
        pltpu.VMEM((64,), jnp.float32),
        pltpu.VMEM((64,), jnp.float32),
        pltpu.VMEM((64 * M,), jnp.float32),
        pltpu.VMEM((64 * M,), jnp.float32),
        pltpu.VMEM((64 * M,), jnp.float32),
    ],
)(_gather_body)


def kernel(xyz):
    x = xyz[:, :, 0]
    y = xyz[:, :, 1]
    z = xyz[:, :, 2]

    cx, cy, cz = pl.pallas_call(
        _fps_body,
        out_shape=[jax.ShapeDtypeStruct((B, G), jnp.float32)] * 3,
    )(x, y, z)

    x3 = x.reshape(B, 1, N)
    y3 = y.reshape(B, 1, N)
    z3 = z.reshape(B, 1, N)
    cxT = cx.reshape(B, G, 1)
    cyT = cy.reshape(B, G, 1)
    czT = cz.reshape(B, G, 1)
    knn, scale = pl.pallas_call(
        _knn_body,
        grid=(B,),
        in_specs=[pl.BlockSpec((1, 1, N), lambda b: (b, 0, 0))] * 3
        + [pl.BlockSpec((1, G, 1), lambda b: (b, 0, 0))] * 3,
        out_specs=[pl.BlockSpec((1, G, M), lambda b: (b, 0, 0)),
                   pl.BlockSpec((1, G, 1), lambda b: (b, 0, 0))],
        out_shape=[jax.ShapeDtypeStruct((B, G, M), jnp.int32),
                   jax.ShapeDtypeStruct((B, G, 1), jnp.float32)],
        scratch_shapes=[pltpu.VMEM((G, N), jnp.float32)],
    )(x3, y3, z3, cxT, cyT, czT)

    ox, oy, oz = _gather_call(
        x.reshape(-1), y.reshape(-1), z.reshape(-1),
        knn.reshape(-1), scale.reshape(-1),
        cx.reshape(-1), cy.reshape(-1), cz.reshape(-1),
    )

    neighborhood = jnp.stack(
        [ox.reshape(B, G, M), oy.reshape(B, G, M), oz.reshape(B, G, M)],
        axis=-1)
    center = jnp.stack([cx, cy, cz], axis=-1)
    return neighborhood, center


# speedup vs baseline: 11.9627x; 1.1547x over previous
import functools

import jax
import jax.numpy as jnp
from jax import lax
from jax.experimental import pallas as pl
from jax.experimental.pallas import tpu as pltpu
from jax.experimental.pallas import tpu_sc as plsc

B, N, G, M = 8, 8192, 256, 32


def _fps_body(x_ref, y_ref, z_ref, cx_ref, cy_ref, cz_ref):
    x = x_ref[...]
    y = y_ref[...]
    z = z_ref[...]
    lane = jax.lax.broadcasted_iota(jnp.int32, (B, N), 1).astype(jnp.float32)
    col = jax.lax.broadcasted_iota(jnp.int32, (B, G), 1)
    fN = jnp.float32(N)

    lastx = x[:, 0:1]
    lasty = y[:, 0:1]
    lastz = z[:, 0:1]
    zerosg = jnp.zeros((B, G), jnp.float32)
    cx = jnp.where(col == 0, lastx, zerosg)
    cy = jnp.where(col == 0, lasty, zerosg)
    cz = jnp.where(col == 0, lastz, zerosg)
    dists = jnp.full((B, N), 1e10, jnp.float32)

    def step(i, carry):
        dists, lastx, lasty, lastz, cx, cy, cz = carry
        dx = x - lastx
        dy = y - lasty
        dz = z - lastz
        d = (dx * dx + dy * dy) + dz * dz
        dists = jnp.minimum(dists, d)
        m = jnp.max(dists, axis=1, keepdims=True)
        t = jnp.where(dists == m, lane, fN)
        idx = jnp.min(t, axis=1, keepdims=True)
        oh = t == idx
        zn = jnp.zeros((B, N), jnp.float32)
        lastx = jnp.sum(jnp.where(oh, x, zn), axis=1, keepdims=True)
        lasty = jnp.sum(jnp.where(oh, y, zn), axis=1, keepdims=True)
        lastz = jnp.sum(jnp.where(oh, z, zn), axis=1, keepdims=True)
        upd = col == i
        cx = jnp.where(upd, lastx, cx)
        cy = jnp.where(upd, lasty, cy)
        cz = jnp.where(upd, lastz, cz)
        return (dists, lastx, lasty, lastz, cx, cy, cz)

    carry = (dists, lastx, lasty, lastz, cx, cy, cz)
    carry = jax.lax.fori_loop(1, G, step, carry)
    _, _, _, _, cx, cy, cz = carry
    cx_ref[...] = cx
    cy_ref[...] = cy
    cz_ref[...] = cz


def _knn_body(x_ref, y_ref, z_ref, cxT_ref, cyT_ref, czT_ref,
              knn_ref, scale_ref, d2_ref):
    x = x_ref[0]
    y = y_ref[0]
    z = z_ref[0]
    cx = cxT_ref[0]
    cy = cyT_ref[0]
    cz = czT_ref[0]

    dx = cx - x
    dy = cy - y
    dz = cz - z
    d2_ref[...] = (dx * dx + dy * dy) + dz * dz

    lane = jax.lax.broadcasted_iota(jnp.int32, (G, N), 1).astype(jnp.float32)
    col = jax.lax.broadcasted_iota(jnp.int32, (G, M), 1)
    inf = jnp.float32(jnp.inf)
    fN = jnp.float32(N)
    zgm = jnp.zeros((G, M), jnp.float32)
    igm = jnp.zeros((G, M), jnp.int32)

    def step(s, carry):
        knn, d2s = carry
        w = d2_ref[...]
        m = jnp.min(w, axis=1, keepdims=True)
        t = jnp.where(w == m, lane, fN)
        idx = jnp.min(t, axis=1, keepdims=True)
        d2_ref[...] = jnp.where(t == idx, inf, w)
        upd = col == s
        knn = jnp.where(upd, idx.astype(jnp.int32), knn)
        d2s = jnp.where(upd, m, d2s)
        return (knn, d2s)

    knn, d2s = jax.lax.fori_loop(0, M, step, (igm, zgm))

    scale = jnp.max(jnp.sqrt(d2s + 1e-12), axis=1, keepdims=True)
    scale = jnp.maximum(scale, 1e-8)
    knn_ref[0] = knn
    scale_ref[0] = scale


def _gather_body(xf, yf, zf, knnf, sf, cxf, cyf, czf,
                 ox_hbm, oy_hbm, oz_hbm,
                 xv, yv, zv, kv, sv, cxv, cyv, czv, oxv, oyv, ozv):
    wid = lax.axis_index("s") * 2 + lax.axis_index("c")
    b = wid // 4
    q = wid % 4
    gbase = b * G + q * 64
    obase = gbase * M

    pltpu.sync_copy(xf.at[pl.ds(b * N, N)], xv)
    pltpu.sync_copy(yf.at[pl.ds(b * N, N)], yv)
    pltpu.sync_copy(zf.at[pl.ds(b * N, N)], zv)
    pltpu.sync_copy(knnf.at[pl.ds(obase, 64 * M)], kv)
    pltpu.sync_copy(sf.at[pl.ds(gbase, 64)], sv)
    pltpu.sync_copy(cxf.at[pl.ds(gbase, 64)], cxv)
    pltpu.sync_copy(cyf.at[pl.ds(gbase, 64)], cyv)
    pltpu.sync_copy(czf.at[pl.ds(gbase, 64)], czv)

    lanes = lax.iota(jnp.int32, 16)
    for gg in range(4):
        cbase = gg * 16
        cxr = cxv[pl.ds(cbase, 16)]
        cyr = cyv[pl.ds(cbase, 16)]
        czr = czv[pl.ds(cbase, 16)]
        sr = sv[pl.ds(cbase, 16)]
        posb = (lanes + cbase) * M
        for j in range(M):
            pos = posb + j
            idx = plsc.load_gather(kv, [pos])
            vx = plsc.load_gather(xv, [idx])
            vy = plsc.load_gather(yv, [idx])
            vz = plsc.load_gather(zv, [idx])
            plsc.store_scatter(oxv, [pos], (vx - cxr) / sr)
            plsc.store_scatter(oyv, [pos], (vy - cyr) / sr)
            plsc.store_scatter(ozv, [pos], (vz - czr) / sr)

    pltpu.sync_copy(oxv, ox_hbm.at[pl.ds(obase, 64 * M)])
    pltpu.sync_copy(oyv, oy_hbm.at[pl.ds(obase, 64 * M)])
    pltpu.sync_copy(ozv, oz_hbm.at[pl.ds(obase, 64 * M)])


_gather_call = functools.partial(
    pl.kernel,
    out_type=[jax.ShapeDtypeStruct((B * G * M,), jnp.float32)] * 3,
    mesh=plsc.VectorSubcoreMesh(core_axis_name="c", subcore_axis_name="s"),
    compiler_params=pltpu.CompilerParams(needs_layout_passes=False),
    scratch_types=[
        pltpu.VMEM((N,), jnp.float32),
        pltpu.VMEM((N,), jnp.float32),
        pltpu.VMEM((N,), jnp.float32),
        pltpu.VMEM((64 * M,), jnp.int32),
        pltpu.VMEM((64,), jnp.float32),
        pltpu.VMEM((64,), jnp.float32),
        pltpu.VMEM((64,), jnp.float32),
        pltpu.VMEM((64,), jnp.float32),
        pltpu.VMEM((64 * M,), jnp.float32),
        pltpu.VMEM((64 * M,), jnp.float32),
        pltpu.VMEM((64 * M,), jnp.float32),
    ],
)(_gather_body)


def kernel(xyz):
    x = xyz[:, :, 0]
    y = xyz[:, :, 1]
    z = xyz[:, :, 2]

    cx, cy, cz = pl.pallas_call(
        _fps_body,
        out_shape=[jax.ShapeDtypeStruct((B, G), jnp.float32)] * 3,
    )(x, y, z)

    x3 = x.reshape(B, 1, N)
    y3 = y.reshape(B, 1, N)
    z3 = z.reshape(B, 1, N)
    cxT = cx.reshape(B, G, 1)
    cyT = cy.reshape(B, G, 1)
    czT = cz.reshape(B, G, 1)
    knn, scale = pl.pallas_call(
        _knn_body,
        grid=(B,),
        in_specs=[pl.BlockSpec((1, 1, N), lambda b: (b, 0, 0))] * 3
        + [pl.BlockSpec((1, G, 1), lambda b: (b, 0, 0))] * 3,
        out_specs=[pl.BlockSpec((1, G, M), lambda b: (b, 0, 0)),
                   pl.BlockSpec((1, G, 1), lambda b: (b, 0, 0))],
        out_shape=[jax.ShapeDtypeStruct((B, G, M), jnp.int32),
                   jax.ShapeDtypeStruct((B, G, 1), jnp.float32)],
        scratch_shapes=[pltpu.VMEM((G, N), jnp.float32)],
    )(x3, y3, z3, cxT, cyT, czT)

    ox, oy, oz = _gather_call(
        x.reshape(-1), y.reshape(-1), z.reshape(-1),
        knn.reshape(-1), scale.reshape(-1),
        cx.reshape(-1), cy.reshape(-1), cz.reshape(-1),
    )

    neighborhood = jnp.stack(
        [ox.reshape(B, G, M), oy.reshape(B, G, M), oz.reshape(B, G, M)],
        axis=-1)
    center = jnp.stack([cx, cy, cz], axis=-1)
    return neighborhood, center
